# baseline (speedup 1.0000x reference)
import functools

import jax
import jax.numpy as jnp
from jax import lax
from jax.experimental import pallas as pl
from jax.experimental.pallas import tpu as pltpu
from jax.experimental.pallas import tpu_sc as plsc

_NC = 2
_NS = 16
_NW = _NC * _NS
_L = 16
_D = 32
_PK = 4


def _mf_kernel(b_per_w, user_idx, item_idx, glob_bas, user_bas, item_bas,
               uv_pack, iv_pack, urow, irow, usub, isub, out, uidx_v, iidx_v,
               urow_v, irow_v, usub_v, isub_v, uv_v, iv_v, ub_v, ib_v, out_v,
               glob_v, sem):
  wid = lax.axis_index("s") * _NC + lax.axis_index("c")
  base = wid * b_per_w
  half = b_per_w // 2

  pltpu.sync_copy(user_idx.at[pl.ds(base, b_per_w)], uidx_v)
  pltpu.sync_copy(item_idx.at[pl.ds(base, b_per_w)], iidx_v)
  pltpu.sync_copy(urow.at[pl.ds(base, b_per_w)], urow_v)
  pltpu.sync_copy(irow.at[pl.ds(base, b_per_w)], irow_v)
  pltpu.sync_copy(usub.at[pl.ds(base, b_per_w)], usub_v)
  pltpu.sync_copy(isub.at[pl.ds(base, b_per_w)], isub_v)
  pltpu.sync_copy(glob_bas, glob_v)

  c_ub = pltpu.async_copy(user_bas.at[uidx_v], ub_v, sem)
  c_ib = pltpu.async_copy(item_bas.at[iidx_v], ib_v, sem)

  gb = glob_v[...]
  lane = lax.iota(jnp.int32, _L)

  def do_half(h):
    hbase = h * half
    c_uv = pltpu.async_copy(uv_pack.at[urow_v.at[pl.ds(hbase, half)]],
                            uv_v, sem)
    c_iv = pltpu.async_copy(iv_pack.at[irow_v.at[pl.ds(hbase, half)]],
                            iv_v, sem)
    c_uv.wait()
    c_iv.wait()

    def group_body(g, _):
      r = g * _L + hbase
      rows = g * _L + lane
      su = usub_v[pl.ds(r, _L)] * _D
      si = isub_v[pl.ds(r, _L)] * _D
      acc = ub_v[pl.ds(r, _L)] + ib_v[pl.ds(r, _L)] + gb
      for c in range(_D):
        u = plsc.load_gather(uv_v, [rows, su + c])
        v = plsc.load_gather(iv_v, [rows, si + c])
        acc = acc + u * v
      out_v[pl.ds(r, _L)] = acc
      return 0

    lax.fori_loop(0, half // _L, group_body, 0)

  c_ub.wait()
  c_ib.wait()
  do_half(0)
  do_half(1)

  pltpu.sync_copy(out_v, out.at[pl.ds(base, b_per_w)])


def kernel(user_idx, item_idx, glob_bas, user_bas, item_bas, user_vec,
           item_vec):
  batch = user_idx.shape[0]
  b_per_w = batch // _NW
  n = user_vec.shape[0]

  uv_pack = user_vec.reshape(n // _PK, _PK * _D)
  iv_pack = item_vec.reshape(n // _PK, _PK * _D)
  uidx = user_idx.astype(jnp.int32)
  iidx = item_idx.astype(jnp.int32)

  mesh = plsc.VectorSubcoreMesh(core_axis_name="c", subcore_axis_name="s",
                                num_cores=_NC, num_subcores=_NS)
  run = pl.kernel(
      functools.partial(_mf_kernel, b_per_w),
      out_type=jax.ShapeDtypeStruct((batch,), jnp.float32),
      mesh=mesh,
      scratch_types=[
          pltpu.VMEM((b_per_w,), jnp.int32),
          pltpu.VMEM((b_per_w,), jnp.int32),
          pltpu.VMEM((b_per_w,), jnp.int32),
          pltpu.VMEM((b_per_w,), jnp.int32),
          pltpu.VMEM((b_per_w,), jnp.int32),
          pltpu.VMEM((b_per_w,), jnp.int32),
          pltpu.VMEM((b_per_w // 2, _PK * _D), jnp.float32),
          pltpu.VMEM((b_per_w // 2, _PK * _D), jnp.float32),
          pltpu.VMEM((b_per_w,), jnp.float32),
          pltpu.VMEM((b_per_w,), jnp.float32),
          pltpu.VMEM((b_per_w,), jnp.float32),
          pltpu.VMEM((_L,), jnp.float32),
          pltpu.SemaphoreType.DMA,
      ],
      compiler_params=pltpu.CompilerParams(use_tc_tiling_on_sc=False,
                                           needs_layout_passes=False),
  )
  glob_b = jnp.broadcast_to(glob_bas.reshape(()), (_L,))
  return run(uidx, iidx, glob_b, user_bas.reshape(-1), item_bas.reshape(-1),
             uv_pack, iv_pack, uidx // _PK, iidx // _PK, uidx % _PK,
             iidx % _PK)


# speedup vs baseline: 1.5682x; 1.5682x over previous
import functools

import jax
import jax.numpy as jnp
from jax import lax
from jax.experimental import pallas as pl
from jax.experimental.pallas import tpu as pltpu
from jax.experimental.pallas import tpu_sc as plsc

_GATHER_DNUMS = lax.GatherDimensionNumbers(
    offset_dims=(), collapsed_slice_dims=(0,), start_index_map=(0,))


def _lane_shuffle(x, idx):
  return lax.gather(x, idx[:, None], dimension_numbers=_GATHER_DNUMS,
                    slice_sizes=(1,),
                    mode=lax.GatherScatterMode.PROMISE_IN_BOUNDS)


_NC = 2
_NS = 16
_NW = _NC * _NS
_L = 16
_D = 32
_TBLK = 20480


def _transpose_body(xu_ref, xi_ref, ou_ref, oi_ref):
  q = _TBLK // 4
  for ref_in, ref_out in ((xu_ref, ou_ref), (xi_ref, oi_ref)):
    parts = [ref_in[:, pl.ds(j * q, q)].T for j in range(4)]
    ref_out[...] = jnp.concatenate(parts, axis=1)


def _retile(user_vecT, item_vecT):
  n = user_vecT.shape[1]
  grid = (pl.cdiv(n, _TBLK),)
  n4 = grid[0] * _TBLK // 4
  out4 = pl.pallas_call(
      _transpose_body,
      grid=grid,
      in_specs=[pl.BlockSpec((_D, _TBLK), lambda i: (0, i)),
                pl.BlockSpec((_D, _TBLK), lambda i: (0, i))],
      out_specs=[pl.BlockSpec((_TBLK // 4, 4 * _D), lambda i: (i, 0)),
                 pl.BlockSpec((_TBLK // 4, 4 * _D), lambda i: (i, 0))],
      out_shape=[jax.ShapeDtypeStruct((n4, 4 * _D), jnp.float32),
                 jax.ShapeDtypeStruct((n4, 4 * _D), jnp.float32)],
  )(user_vecT, item_vecT)
  return [o.reshape(-1, _D) for o in out4]


def _mf_kernel(b_per_w, user_idx, item_idx, glob_bas, user_bas, item_bas,
               user_vec, item_vec, puidx, piidx, out, uidx_v, iidx_v, pu_v,
               pi_v, uv_v, iv_v, ub_v, ib_v, out_v, glob_v, sem):
  wid = lax.axis_index("s") * _NC + lax.axis_index("c")
  base = wid * b_per_w

  pltpu.sync_copy(user_idx.at[pl.ds(base, b_per_w)], uidx_v)
  pltpu.sync_copy(item_idx.at[pl.ds(base, b_per_w)], iidx_v)
  pltpu.sync_copy(puidx.at[pl.ds(base, b_per_w)], pu_v)
  pltpu.sync_copy(piidx.at[pl.ds(base, b_per_w)], pi_v)
  pltpu.sync_copy(glob_bas, glob_v)

  c_uv = pltpu.async_copy(user_vec.at[pu_v], uv_v, sem)
  c_iv = pltpu.async_copy(item_vec.at[pi_v], iv_v, sem)
  c_ub = pltpu.async_copy(user_bas.at[uidx_v], ub_v, sem)
  c_ib = pltpu.async_copy(item_bas.at[iidx_v], ib_v, sem)
  c_uv.wait()
  c_iv.wait()
  c_ub.wait()
  c_ib.wait()

  gb = glob_v[...]
  lane = lax.iota(jnp.int32, _L)

  def group_body(g, _):
    acc = ub_v[pl.ds(g * _L, _L)] + ib_v[pl.ds(g * _L, _L)] + gb
    for j in range(_L):
      r = g * _L + j
      u0 = uv_v[r, pl.ds(0, _L)]
      u1 = uv_v[r, pl.ds(_L, _L)]
      v0 = iv_v[r, pl.ds(0, _L)]
      v1 = iv_v[r, pl.ds(_L, _L)]
      p = u0 * v0 + u1 * v1
      for d in (8, 4, 2, 1):
        p = p + _lane_shuffle(p, lane ^ d)
      acc = jnp.where(lane == j, acc + p, acc)
    out_v[pl.ds(g * _L, _L)] = acc
    return 0

  lax.fori_loop(0, b_per_w // _L, group_body, 0)

  pltpu.sync_copy(out_v, out.at[pl.ds(base, b_per_w)])


def kernel(user_idx, item_idx, glob_bas, user_bas, item_bas, user_vec,
           item_vec):
  batch = user_idx.shape[0]
  b_per_w = batch // _NW

  uv_rm, iv_rm = _retile(user_vec.T, item_vec.T)

  q = _TBLK // 4
  def _remap(idx):
    idx = idx.astype(jnp.int32)
    b, w = idx // _TBLK, idx % _TBLK
    return ((b * q + w % q) * 4 + w // q).astype(jnp.int32)

  mesh = plsc.VectorSubcoreMesh(core_axis_name="c", subcore_axis_name="s",
                                num_cores=_NC, num_subcores=_NS)
  run = pl.kernel(
      functools.partial(_mf_kernel, b_per_w),
      out_type=jax.ShapeDtypeStruct((batch,), jnp.float32),
      mesh=mesh,
      scratch_types=[
          pltpu.VMEM((b_per_w,), jnp.int32),
          pltpu.VMEM((b_per_w,), jnp.int32),
          pltpu.VMEM((b_per_w,), jnp.int32),
          pltpu.VMEM((b_per_w,), jnp.int32),
          pltpu.VMEM((b_per_w, _D), jnp.float32),
          pltpu.VMEM((b_per_w, _D), jnp.float32),
          pltpu.VMEM((b_per_w,), jnp.float32),
          pltpu.VMEM((b_per_w,), jnp.float32),
          pltpu.VMEM((b_per_w,), jnp.float32),
          pltpu.VMEM((_L,), jnp.float32),
          pltpu.SemaphoreType.DMA,
      ],
      compiler_params=pltpu.CompilerParams(use_tc_tiling_on_sc=False),
  )
  glob_b = jnp.broadcast_to(glob_bas.reshape(()), (_L,))
  return run(user_idx.astype(jnp.int32), item_idx.astype(jnp.int32),
             glob_b, user_bas.reshape(-1), item_bas.reshape(-1),
             uv_rm, iv_rm, _remap(user_idx), _remap(item_idx))
